# baseline (speedup 1.0000x reference)
import functools

import jax
import jax.numpy as jnp
from jax import lax
from jax.experimental import pallas as pl
from jax.experimental.pallas import tpu as pltpu
from jax.experimental.pallas import tpu_sc as plsc

B, W, H, C = 8, 224, 224, 96
HC = H * C
LANES = 16
UNROLL = 8
NSTEP = HC // (LANES * UNROLL)
NWORK = 32
NTASK = B * C
TPW = NTASK // NWORK

_mesh = plsc.VectorSubcoreMesh(core_axis_name="c", subcore_axis_name="s")


@functools.partial(
    pl.kernel,
    mesh=_mesh,
    out_type=jax.ShapeDtypeStruct((NTASK * 2,), jnp.float32),
    scratch_types=[
        pltpu.VMEM((HC,), jnp.float32),
        pltpu.VMEM((HC,), jnp.float32),
        pltpu.VMEM((2 * TPW,), jnp.float32),
        pltpu.SemaphoreType.DMA,
        pltpu.SemaphoreType.DMA,
    ],
)
def _argmax_sc(x_hbm, out_hbm, buf0, buf1, obuf, sem0, sem1):
    cid = lax.axis_index("c")
    sid = lax.axis_index("s")
    wid = sid * 2 + cid
    t0 = wid * TPW

    bufs = (buf0, buf1)
    sems = (sem0, sem1)

    def src_row(t):
        return (t // C) * W + (t % C)

    copies = [pltpu.async_copy(x_hbm.at[src_row(t0)], buf0, sem0), None]
    lanes = lax.iota(jnp.int32, LANES)
    acc = jnp.zeros((LANES,), jnp.float32)

    for k in range(TPW):
        if k + 1 < TPW:
            nb = (k + 1) % 2
            copies[nb] = pltpu.async_copy(
                x_hbm.at[src_row(t0 + k + 1)], bufs[nb], sems[nb])
        copies[k % 2].wait()
        buf = bufs[k % 2]

        def step(c, carry):
            m, rk = carry
            base = c * (LANES * UNROLL)
            for u in range(UNROLL):
                v = buf[pl.ds(base + u * LANES, LANES)]
                gt = v > m
                m = jnp.where(gt, v, m)
                rk = jnp.where(gt, c * UNROLL + u, rk)
            return m, rk

        m0 = jnp.full((LANES,), -jnp.inf, jnp.float32)
        rk0 = jnp.zeros((LANES,), jnp.int32)
        m, rk = lax.fori_loop(0, NSTEP, step, (m0, rk0))

        idx = rk * LANES + lanes
        bv, bi = m[0], idx[0]
        for l in range(1, LANES):
            v, i = m[l], idx[l]
            take = jnp.logical_or(v > bv, jnp.logical_and(v == bv, i < bi))
            bv = jnp.where(take, v, bv)
            bi = jnp.where(take, i, bi)
        gidx = bi
        colf = (gidx % W).astype(jnp.float32)
        rowf = (gidx // W).astype(jnp.float32)
        p = (2 * k) % LANES
        acc = jnp.where(lanes == p, colf, acc)
        acc = jnp.where(lanes == p + 1, rowf, acc)
        if p + 2 == LANES:
            obuf[pl.ds((k // (LANES // 2)) * LANES, LANES)] = acc

    pltpu.sync_copy(obuf, out_hbm.at[pl.ds(t0 * 2, 2 * TPW)])


def kernel(x):
    out = _argmax_sc(x.reshape(B * W, HC))
    return out.reshape(B, 2 * C)


# speedup vs baseline: 1.0111x; 1.0111x over previous
import functools

import jax
import jax.numpy as jnp
from jax import lax
from jax.experimental import pallas as pl
from jax.experimental.pallas import tpu as pltpu
from jax.experimental.pallas import tpu_sc as plsc

B, W, H, C = 8, 224, 224, 96
HC = H * C
LANES = 16
UNROLL = 8
NSTEP = HC // (LANES * UNROLL)
NWORK = 32
NTASK = B * C
TPW = NTASK // NWORK

_mesh = plsc.VectorSubcoreMesh(core_axis_name="c", subcore_axis_name="s")


@functools.partial(
    pl.kernel,
    mesh=_mesh,
    out_type=jax.ShapeDtypeStruct((NTASK * 2,), jnp.float32),
    scratch_types=[
        pltpu.VMEM((HC,), jnp.float32),
        pltpu.VMEM((HC,), jnp.float32),
        pltpu.VMEM((2 * TPW,), jnp.float32),
        pltpu.SemaphoreType.DMA,
        pltpu.SemaphoreType.DMA,
    ],
)
def _argmax_sc(x_hbm, out_hbm, buf0, buf1, obuf, sem0, sem1):
    cid = lax.axis_index("c")
    sid = lax.axis_index("s")
    wid = sid * 2 + cid
    t0 = wid * TPW

    bufs = (buf0, buf1)
    sems = (sem0, sem1)

    def src_row(t):
        return (t // C) * W + (t % C)

    copies = [pltpu.async_copy(x_hbm.at[src_row(t0)], buf0, sem0), None]
    lanes = lax.iota(jnp.int32, LANES)
    acc = jnp.zeros((LANES,), jnp.float32)

    for k in range(TPW):
        if k + 1 < TPW:
            nb = (k + 1) % 2
            copies[nb] = pltpu.async_copy(
                x_hbm.at[src_row(t0 + k + 1)], bufs[nb], sems[nb])
        copies[k % 2].wait()
        buf = bufs[k % 2]

        def step(c, carry):
            new = []
            base = c * (LANES * UNROLL)
            for u in range(UNROLL):
                m, rk = carry[u]
                v = buf[pl.ds(base + u * LANES, LANES)]
                gt = v > m
                m = jnp.where(gt, v, m)
                rk = jnp.where(gt, c * UNROLL + u, rk)
                new.append((m, rk))
            return tuple(new)

        m0 = jnp.full((LANES,), -jnp.inf, jnp.float32)
        rk0 = jnp.zeros((LANES,), jnp.int32)
        accs = lax.fori_loop(0, NSTEP, step, tuple((m0, rk0) for _ in range(UNROLL)))

        m, rk = accs[0]
        for u in range(1, UNROLL):
            mu, rku = accs[u]
            take = jnp.logical_or(mu > m, jnp.logical_and(mu == m, rku < rk))
            m = jnp.where(take, mu, m)
            rk = jnp.where(take, rku, rk)

        idx = rk * LANES + lanes
        bv, bi = m[0], idx[0]
        for l in range(1, LANES):
            v, i = m[l], idx[l]
            take = jnp.logical_or(v > bv, jnp.logical_and(v == bv, i < bi))
            bv = jnp.where(take, v, bv)
            bi = jnp.where(take, i, bi)
        gidx = bi
        colf = (gidx % W).astype(jnp.float32)
        rowf = (gidx // W).astype(jnp.float32)
        p = (2 * k) % LANES
        acc = jnp.where(lanes == p, colf, acc)
        acc = jnp.where(lanes == p + 1, rowf, acc)
        if p + 2 == LANES:
            obuf[pl.ds((k // (LANES // 2)) * LANES, LANES)] = acc

    pltpu.sync_copy(obuf, out_hbm.at[pl.ds(t0 * 2, 2 * TPW)])


def kernel(x):
    out = _argmax_sc(x.reshape(B * W, HC))
    return out.reshape(B, 2 * C)


# speedup vs baseline: 3.9354x; 3.8921x over previous
import functools

import jax
import jax.numpy as jnp
from jax import lax
from jax.experimental import pallas as pl
from jax.experimental.pallas import tpu as pltpu
from jax.experimental.pallas import tpu_sc as plsc

B, W, H, C = 8, 224, 224, 96
HC = H * C
LANES = 16
CCHUNK = C // LANES
NWORK = 32
NTASK = B * C
TPW = NTASK // NWORK

_mesh = plsc.VectorSubcoreMesh(core_axis_name="c", subcore_axis_name="s")


@functools.partial(
    pl.kernel,
    mesh=_mesh,
    out_type=jax.ShapeDtypeStruct((NTASK * 2,), jnp.float32),
    scratch_types=[
        pltpu.VMEM((H, C), jnp.float32),
        pltpu.VMEM((H, C), jnp.float32),
        pltpu.VMEM((2 * TPW,), jnp.float32),
        pltpu.SemaphoreType.DMA,
        pltpu.SemaphoreType.DMA,
    ],
)
def _argmax_sc(x_hbm, out_hbm, buf0, buf1, obuf, sem0, sem1):
    cid = lax.axis_index("c")
    sid = lax.axis_index("s")
    wid = sid * 2 + cid
    t0 = wid * TPW

    bufs = (buf0, buf1)
    sems = (sem0, sem1)

    def start_copy(k):
        t = t0 + k
        return pltpu.async_copy(x_hbm.at[t // C, t % C], bufs[k % 2], sems[k % 2])

    copies = [start_copy(0), None]
    lanes = lax.iota(jnp.int32, LANES)
    acc = jnp.zeros((LANES,), jnp.float32)

    for k in range(TPW):
        if k + 1 < TPW:
            copies[(k + 1) % 2] = start_copy(k + 1)
        copies[k % 2].wait()
        buf = bufs[k % 2]

        def step(h, carry):
            new = []
            for u in range(CCHUNK):
                m, rk = carry[u]
                v = buf[h, pl.ds(u * LANES, LANES)]
                gt = v > m
                m = jnp.where(gt, v, m)
                rk = jnp.where(gt, h * CCHUNK + u, rk)
                new.append((m, rk))
            return tuple(new)

        m0 = jnp.full((LANES,), -jnp.inf, jnp.float32)
        rk0 = jnp.zeros((LANES,), jnp.int32)
        accs = lax.fori_loop(0, H, step, tuple((m0, rk0) for _ in range(CCHUNK)))

        m, rk = accs[0]
        for u in range(1, CCHUNK):
            mu, rku = accs[u]
            take = jnp.logical_or(mu > m, jnp.logical_and(mu == m, rku < rk))
            m = jnp.where(take, mu, m)
            rk = jnp.where(take, rku, rk)

        idx = rk * LANES + lanes
        bv, bi = m[0], idx[0]
        for l in range(1, LANES):
            v, i = m[l], idx[l]
            take = jnp.logical_or(v > bv, jnp.logical_and(v == bv, i < bi))
            bv = jnp.where(take, v, bv)
            bi = jnp.where(take, i, bi)
        gidx = bi
        colf = (gidx % W).astype(jnp.float32)
        rowf = (gidx // W).astype(jnp.float32)
        p = (2 * k) % LANES
        acc = jnp.where(lanes == p, colf, acc)
        acc = jnp.where(lanes == p + 1, rowf, acc)
        if p + 2 == LANES:
            obuf[pl.ds((k // (LANES // 2)) * LANES, LANES)] = acc

    pltpu.sync_copy(obuf, out_hbm.at[pl.ds(t0 * 2, 2 * TPW)])


def kernel(x):
    out = _argmax_sc(x)
    return out.reshape(B, 2 * C)


# speedup vs baseline: 4.9160x; 1.2492x over previous
import functools

import jax
import jax.numpy as jnp
from jax import lax
from jax.experimental import pallas as pl
from jax.experimental.pallas import tpu as pltpu
from jax.experimental.pallas import tpu_sc as plsc

B, W, H, C = 8, 224, 224, 96
HC = H * C
LANES = 16
CCHUNK = C // LANES
NWORK = 32
NTASK = B * C
TPW = NTASK // NWORK

_mesh = plsc.VectorSubcoreMesh(core_axis_name="c", subcore_axis_name="s")


@functools.partial(
    pl.kernel,
    mesh=_mesh,
    out_type=jax.ShapeDtypeStruct((NTASK * 2,), jnp.float32),
    scratch_types=[
        pltpu.VMEM((H, C), jnp.float32),
        pltpu.VMEM((H, C), jnp.float32),
        pltpu.VMEM((2 * TPW,), jnp.float32),
        pltpu.SemaphoreType.DMA,
        pltpu.SemaphoreType.DMA,
    ],
)
def _argmax_sc(x_hbm, out_hbm, buf0, buf1, obuf, sem0, sem1):
    cid = lax.axis_index("c")
    sid = lax.axis_index("s")
    wid = sid * 2 + cid
    t0 = wid * TPW

    bufs = (buf0, buf1)
    sems = (sem0, sem1)

    def start_copy(k):
        t = t0 + k
        return pltpu.async_copy(x_hbm.at[t // C, t % C], bufs[k % 2], sems[k % 2])

    copies = [start_copy(0), None]
    lanes = lax.iota(jnp.int32, LANES)
    acc = jnp.zeros((LANES,), jnp.float32)

    for k in range(TPW):
        if k + 1 < TPW:
            copies[(k + 1) % 2] = start_copy(k + 1)
        copies[k % 2].wait()
        buf = bufs[k % 2]

        def step(h, carry):
            new = []
            for u in range(CCHUNK):
                m, rk = carry[u]
                v = buf[h, pl.ds(u * LANES, LANES)]
                gt = v > m
                m = jnp.where(gt, v, m)
                rk = jnp.where(gt, h * CCHUNK + u, rk)
                new.append((m, rk))
            return tuple(new)

        m0 = jnp.full((LANES,), -jnp.inf, jnp.float32)
        rk0 = jnp.zeros((LANES,), jnp.int32)
        accs = lax.fori_loop(0, H, step, tuple((m0, rk0) for _ in range(CCHUNK)))

        m, rk = accs[0]
        for u in range(1, CCHUNK):
            mu, rku = accs[u]
            take = jnp.logical_or(mu > m, jnp.logical_and(mu == m, rku < rk))
            m = jnp.where(take, mu, m)
            rk = jnp.where(take, rku, rk)

        idx = rk * LANES + lanes
        bv, bi = m[0], idx[0]
        for l in range(1, LANES):
            v, i = m[l], idx[l]
            take = jnp.logical_or(v > bv, jnp.logical_and(v == bv, i < bi))
            bv = jnp.where(take, v, bv)
            bi = jnp.where(take, i, bi)
        gidx = bi
        colf = (gidx % W).astype(jnp.float32)
        rowf = (gidx // W).astype(jnp.float32)
        p = (2 * k) % LANES
        acc = jnp.where(lanes == p, colf, acc)
        acc = jnp.where(lanes == p + 1, rowf, acc)
        if p + 2 == LANES:
            obuf[pl.ds((k // (LANES // 2)) * LANES, LANES)] = acc

    pltpu.sync_copy(obuf, out_hbm.at[pl.ds(t0 * 2, 2 * TPW)])


def kernel(x):
    out = _argmax_sc(x[:, :C])
    return out.reshape(B, 2 * C)


# speedup vs baseline: 4.9194x; 1.0007x over previous
import functools

import jax
import jax.numpy as jnp
from jax import lax
from jax.experimental import pallas as pl
from jax.experimental.pallas import tpu as pltpu
from jax.experimental.pallas import tpu_sc as plsc

B, W, H, C = 8, 224, 224, 96
HC = H * C
LANES = 16
CCHUNK = C // LANES
NWORK = 32
NTASK = B * C
TPW = NTASK // NWORK

_mesh = plsc.VectorSubcoreMesh(core_axis_name="c", subcore_axis_name="s")


@functools.partial(
    pl.kernel,
    mesh=_mesh,
    out_type=jax.ShapeDtypeStruct((NTASK * 2,), jnp.float32),
    scratch_types=[
        pltpu.VMEM((H, C), jnp.float32),
        pltpu.VMEM((H, C), jnp.float32),
        pltpu.VMEM((2 * TPW,), jnp.float32),
        pltpu.SemaphoreType.DMA,
        pltpu.SemaphoreType.DMA,
    ],
    compiler_params=pltpu.CompilerParams(use_tc_tiling_on_sc=True),
)
def _argmax_sc(x_hbm, out_hbm, buf0, buf1, obuf, sem0, sem1):
    cid = lax.axis_index("c")
    sid = lax.axis_index("s")
    wid = sid * 2 + cid
    t0 = wid * TPW

    bufs = (buf0, buf1)
    sems = (sem0, sem1)

    def start_copy(k):
        t = t0 + k
        return pltpu.async_copy(x_hbm.at[t // C, t % C], bufs[k % 2], sems[k % 2])

    copies = [start_copy(0), None]
    lanes = lax.iota(jnp.int32, LANES)
    acc = jnp.zeros((LANES,), jnp.float32)

    for k in range(TPW):
        if k + 1 < TPW:
            copies[(k + 1) % 2] = start_copy(k + 1)
        copies[k % 2].wait()
        buf = bufs[k % 2]

        def step(h, carry):
            new = []
            for u in range(CCHUNK):
                m, rk = carry[u]
                v = buf[h, pl.ds(u * LANES, LANES)]
                gt = v > m
                m = jnp.where(gt, v, m)
                rk = jnp.where(gt, h * CCHUNK + u, rk)
                new.append((m, rk))
            return tuple(new)

        m0 = jnp.full((LANES,), -jnp.inf, jnp.float32)
        rk0 = jnp.zeros((LANES,), jnp.int32)
        accs = lax.fori_loop(0, H, step, tuple((m0, rk0) for _ in range(CCHUNK)))

        m, rk = accs[0]
        for u in range(1, CCHUNK):
            mu, rku = accs[u]
            take = jnp.logical_or(mu > m, jnp.logical_and(mu == m, rku < rk))
            m = jnp.where(take, mu, m)
            rk = jnp.where(take, rku, rk)

        idx = rk * LANES + lanes
        bv, bi = m[0], idx[0]
        for l in range(1, LANES):
            v, i = m[l], idx[l]
            take = jnp.logical_or(v > bv, jnp.logical_and(v == bv, i < bi))
            bv = jnp.where(take, v, bv)
            bi = jnp.where(take, i, bi)
        gidx = bi
        colf = (gidx % W).astype(jnp.float32)
        rowf = (gidx // W).astype(jnp.float32)
        p = (2 * k) % LANES
        acc = jnp.where(lanes == p, colf, acc)
        acc = jnp.where(lanes == p + 1, rowf, acc)
        if p + 2 == LANES:
            obuf[pl.ds((k // (LANES // 2)) * LANES, LANES)] = acc

    pltpu.sync_copy(obuf, out_hbm.at[pl.ds(t0 * 2, 2 * TPW)])


def kernel(x):
    out = _argmax_sc(x[:, :C])
    return out.reshape(B, 2 * C)
